# baseline (speedup 1.0000x reference)
import functools

import jax
import jax.numpy as jnp
from jax import lax
from jax.experimental import pallas as pl
from jax.experimental.pallas import tpu as pltpu
from jax.experimental.pallas import tpu_sc as plsc

N = 10000
NP = 10240
IN = 128
H1 = 8
HID = 8
F1 = H1 * HID
OUT = 16
NC = 2
NS = 16
NW = NC * NS
ROWS_PER_SC = NP // NS
C = 512


def _lrelu(x, slope):
  return jnp.where(x > 0, x, x * slope)


def _tc1_body(x_ref, w1_ref, asrc_w_ref, adst_w_ref, p_ref,
              asrc_o, adst_o, m_o, xpp_o):
  x = x_ref[...]
  xp = lax.dot_general(x, w1_ref[...], (((1,), (1,)), ((), ())),
                       preferred_element_type=jnp.float32)
  asrc = jnp.dot(xp, asrc_w_ref[...], preferred_element_type=jnp.float32)
  adst = jnp.dot(xp, adst_w_ref[...], preferred_element_type=jnp.float32)
  gmax = jnp.max(asrc, axis=0, keepdims=True)
  m = _lrelu(gmax + adst, 0.2)
  xpp = jnp.dot(xp, p_ref[...], preferred_element_type=jnp.float32)
  zpad8 = jnp.zeros((NP - N, H1), jnp.float32)
  asrc_o[...] = jnp.concatenate([asrc, zpad8], axis=0)
  adst_o[...] = jnp.concatenate([adst, zpad8], axis=0)
  m_o[...] = jnp.concatenate([m, zpad8], axis=0)
  xpp_b = jnp.concatenate([xpp, jnp.zeros((NP - N, F1), jnp.float32)],
                          axis=0)
  xpp_o[...] = xpp_b.astype(jnp.bfloat16)


def _tc2_body(outp_ref, b1_ref, w2_ref, as2_ref, ad2_ref, p_ref,
              xp2_o, as2_o, ad2_o, m2_o):
  s = outp_ref[0:NP, :] + outp_ref[NP:2 * NP, :]
  o1 = lax.dot_general(s, p_ref[...], (((1,), (1,)), ((), ())),
                       preferred_element_type=jnp.float32)
  h = _lrelu(o1 + b1_ref[...], 0.01)
  row = lax.broadcasted_iota(jnp.int32, (NP, 1), 0)
  h = jnp.where(row < N, h, 0.0)
  xp2 = lax.dot_general(h, w2_ref[...], (((1,), (1,)), ((), ())),
                        preferred_element_type=jnp.float32)
  asrc2 = lax.dot_general(as2_ref[...], xp2, (((1,), (1,)), ((), ())),
                          preferred_element_type=jnp.float32)
  adst2 = lax.dot_general(ad2_ref[...], xp2, (((1,), (1,)), ((), ())),
                          preferred_element_type=jnp.float32)
  gmax2 = jnp.max(asrc2)
  m2 = _lrelu(gmax2 + adst2, 0.2)
  xp2_o[...] = xp2
  as2_o[...] = asrc2
  ad2_o[...] = adst2
  m2_o[...] = m2


def _tc3_body(outp_ref, b2_ref, o_ref):
  o_ref[...] = (outp_ref[0:N, :] + outp_ref[NP:NP + N, :]) + b2_ref[...]


def _stage_slice(hbm_ref, spmem_ref, sid):
  r0 = sid * ROWS_PER_SC
  pltpu.sync_copy(hbm_ref.at[pl.ds(r0, ROWS_PER_SC)],
                  spmem_ref.at[pl.ds(r0, ROWS_PER_SC)])


def _combine_den(den_h, s_den, sid, buf_a, buf_b, F):
  r0 = sid * ROWS_PER_SC
  for i in range(ROWS_PER_SC // 128):
    rr = r0 + i * 128
    pltpu.sync_copy(den_h.at[pl.ds(rr, 128)], buf_a.at[pl.ds(0, 128)])
    pltpu.sync_copy(den_h.at[pl.ds(NP + rr, 128)], buf_b.at[pl.ds(0, 128)])

    def dblk(j, c2):
      sl = pl.ds(j * (128 // F), 128 // F)
      buf_a[sl, :] = buf_a[sl, :] + buf_b[sl, :]
      return c2

    lax.fori_loop(0, 128 // (128 // F), dblk, 0)
    pltpu.sync_copy(buf_a.at[pl.ds(0, 128)], s_den.at[pl.ds(rr, 128)])
  plsc.subcore_barrier()


def _sc_pass1_body(Ck, n_chunks, F, write_ex, asrc_h, adst_h, m_h, src_h,
                   dst_h, z_h,
                   *refs):
  if write_ex:
    den_o, ex_o = refs[0], refs[1]
    rest = refs[2:]
  else:
    den_o = refs[0]
    rest = refs[1:]
  (s_asrc, s_adst, s_m, s_den,
   v_src, v_dst, r_asrc, r_adst, r_m, r_ex,
   sem0, sem1, sem2, sem3, sem4) = rest
  cid = lax.axis_index("c")
  sid = lax.axis_index("s")
  wid = sid * NC + cid
  _stage_slice(asrc_h, s_asrc, sid)
  _stage_slice(adst_h, s_adst, sid)
  _stage_slice(m_h, s_m, sid)
  _stage_slice(z_h, s_den, sid)
  plsc.subcore_barrier()
  base = wid * (n_chunks * Ck)
  B = 128 // F

  def chunk(ch, carry):
    eoff = base + ch * Ck
    d0 = pltpu.async_copy(src_h.at[pl.ds(eoff, Ck)], v_src, sem0)
    d1 = pltpu.async_copy(dst_h.at[pl.ds(eoff, Ck)], v_dst, sem1)
    d0.wait()
    d2 = pltpu.async_copy(s_asrc.at[v_src], r_asrc, sem2)
    d1.wait()
    d3 = pltpu.async_copy(s_adst.at[v_dst], r_adst, sem3)
    d4 = pltpu.async_copy(s_m.at[v_dst], r_m, sem4)
    d2.wait()
    d3.wait()
    d4.wait()

    def blk(j, c2):
      sl = pl.ds(j * B, B)
      a = r_asrc[sl, :] + r_adst[sl, :]
      r_ex[sl, :] = jnp.exp(_lrelu(a, 0.2) - r_m[sl, :])
      return c2

    lax.fori_loop(0, Ck // B, blk, 0)
    pltpu.sync_copy(r_ex, s_den.at[v_dst], add=True)
    if write_ex:
      pltpu.sync_copy(r_ex, ex_o.at[pl.ds(eoff, Ck)])
    return carry

  lax.fori_loop(0, n_chunks, chunk, 0)
  plsc.subcore_barrier()
  r0 = sid * ROWS_PER_SC
  pltpu.sync_copy(s_den.at[pl.ds(r0, ROWS_PER_SC)],
                  den_o.at[pl.ds(cid * NP + r0, ROWS_PER_SC)])


def _sc_pass2_body(Ck, n_chunks, den_h, xpp_h, ex_h, src_h, dst_h, z64_h,
                   out_o,
                   s_den, s_out,
                   v_srcA, v_dstA, r_exA, r_denA, r_xpA, r_msgA,
                   v_srcB, v_dstB, r_exB, r_denB, r_xpB, r_msgB,
                   sA0, sA1, sA2, sA3, sA4, sSA,
                   sB0, sB1, sB2, sB3, sB4, sSB):
  cid = lax.axis_index("c")
  sid = lax.axis_index("s")
  wid = sid * NC + cid
  r0 = sid * ROWS_PER_SC
  _stage_slice(z64_h, s_out, sid)
  _combine_den(den_h, s_den, sid, r_exA, r_denA, H1)
  base = wid * (n_chunks * Ck)

  def issue_a(eoff):
    pltpu.sync_copy(src_h.at[pl.ds(eoff, Ck)], v_srcA)
    pltpu.sync_copy(dst_h.at[pl.ds(eoff, Ck)], v_dstA)
    pltpu.async_copy(xpp_h.at[v_srcA], r_xpA, sA2)
    pltpu.async_copy(ex_h.at[pl.ds(eoff, Ck)], r_exA, sA4)
    pltpu.async_copy(s_den.at[v_dstA], r_denA, sA3)

  def wait_a(eoff):
    pltpu.make_async_copy(xpp_h.at[v_srcA], r_xpA, sA2).wait()
    pltpu.make_async_copy(ex_h.at[pl.ds(eoff, Ck)], r_exA, sA4).wait()
    pltpu.make_async_copy(s_den.at[v_dstA], r_denA, sA3).wait()

  def compute(r_ex, r_den, r_xp, r_msg):
    def cblk(j, c2):
      sl = pl.ds(j * 16, 16)
      r_den[sl, :] = r_ex[sl, :] / (r_den[sl, :] + 1e-16)
      return c2

    lax.fori_loop(0, Ck // 16, cblk, 0)

    def blk(j, c2):
      sl = pl.ds(j * 16, 16)
      cf = r_den[sl, :]
      for c in range(HID):
        csl = pl.ds(c * H1, H1)
        r_msg[sl, csl] = r_xp[sl, csl].astype(jnp.float32) * cf
      return c2

    lax.fori_loop(0, Ck // 16, blk, 0)

  issue_a(base)

  def pair(i, carry):
    ea = base + 2 * i * Ck
    eb = ea + Ck
    en = ea + 2 * Ck
    db0 = pltpu.async_copy(src_h.at[pl.ds(eb, Ck)], v_srcB, sB0)
    db1 = pltpu.async_copy(dst_h.at[pl.ds(eb, Ck)], v_dstB, sB1)
    wait_a(ea)
    db0.wait()
    gb2 = pltpu.async_copy(xpp_h.at[v_srcB], r_xpB, sB2)
    gb4 = pltpu.async_copy(ex_h.at[pl.ds(eb, Ck)], r_exB, sB4)
    db1.wait()
    gb3 = pltpu.async_copy(s_den.at[v_dstB], r_denB, sB3)
    compute(r_exA, r_denA, r_xpA, r_msgA)
    sca = pltpu.async_copy(r_msgA, s_out.at[v_dstA], sSA, add=True)
    gb2.wait()
    gb3.wait()
    gb4.wait()
    compute(r_exB, r_denB, r_xpB, r_msgB)
    scb = pltpu.async_copy(r_msgB, s_out.at[v_dstB], sSB, add=True)
    sca.wait()
    issue_a(en)
    scb.wait()
    return carry

  lax.fori_loop(0, n_chunks // 2, pair, 0)
  wait_a(base + n_chunks * Ck)
  plsc.subcore_barrier()
  pltpu.sync_copy(s_out.at[pl.ds(r0, ROWS_PER_SC)],
                  out_o.at[pl.ds(cid * NP + r0, ROWS_PER_SC)])


def _sc2_pass1_body(Ck, n_chunks, as2_h, ad2_h, m2_h, src_h, dst_h, z1_h,
                    den_o,
                    s_as, s_ad, s_m, s_den,
                    v_src, v_dst, r_as, r_ad, r_m, r_ex,
                    sem0, sem1, sem2, sem3, sem4):
  cid = lax.axis_index("c")
  sid = lax.axis_index("s")
  wid = sid * NC + cid
  _stage_slice(as2_h, s_as, sid)
  _stage_slice(ad2_h, s_ad, sid)
  _stage_slice(m2_h, s_m, sid)
  _stage_slice(z1_h, s_den, sid)
  plsc.subcore_barrier()
  base = wid * (n_chunks * Ck)

  def chunk(ch, carry):
    eoff = base + ch * Ck
    d0 = pltpu.async_copy(src_h.at[pl.ds(eoff, Ck)], v_src, sem0)
    d1 = pltpu.async_copy(dst_h.at[pl.ds(eoff, Ck)], v_dst, sem1)
    d0.wait()
    d2 = pltpu.async_copy(s_as.at[v_src], r_as, sem2)
    d1.wait()
    d3 = pltpu.async_copy(s_ad.at[v_dst], r_ad, sem3)
    d4 = pltpu.async_copy(s_m.at[v_dst], r_m, sem4)
    d2.wait()
    d3.wait()
    d4.wait()

    def blk(j, c2):
      sl = pl.ds(j * 128, 128)
      a = r_as[sl] + r_ad[sl]
      r_ex[sl] = jnp.exp(_lrelu(a, 0.2) - r_m[sl])
      return c2

    lax.fori_loop(0, Ck // 128, blk, 0)
    pltpu.sync_copy(r_ex, s_den.at[v_dst], add=True)
    return carry

  lax.fori_loop(0, n_chunks, chunk, 0)
  plsc.subcore_barrier()
  r0 = sid * ROWS_PER_SC
  pltpu.sync_copy(s_den.at[pl.ds(r0, ROWS_PER_SC)],
                  den_o.at[pl.ds(cid * NP + r0, ROWS_PER_SC)])


def _sc2_pass2_body(Ck, n_chunks, as2_h, ad2_h, m2_h, den_h, xp2_h, src_h,
                    dst_h, z16_h,
                    out_o,
                    s_as, s_ad, s_m, s_den, s_out,
                    v_srcA, v_dstA, r_asA, r_adA, r_mA, r_denA, r_xpA,
                    v_srcB, v_dstB, r_asB, r_adB, r_mB, r_denB, r_xpB,
                    sA0, sA1, sA2, sA3, sA4, sA5, sSA,
                    sB0, sB1, sB2, sB3, sB4, sB5, sSB):
  cid = lax.axis_index("c")
  sid = lax.axis_index("s")
  wid = sid * NC + cid
  r0 = sid * ROWS_PER_SC
  _stage_slice(as2_h, s_as, sid)
  _stage_slice(ad2_h, s_ad, sid)
  _stage_slice(m2_h, s_m, sid)
  _stage_slice(z16_h, s_out, sid)
  for i in range(ROWS_PER_SC // 128):
    rr = r0 + i * 128
    pltpu.sync_copy(den_h.at[pl.ds(rr, 128)], r_asA.at[pl.ds(0, 128)])
    pltpu.sync_copy(den_h.at[pl.ds(NP + rr, 128)], r_adA.at[pl.ds(0, 128)])
    r_asA[pl.ds(0, 128)] = r_asA[pl.ds(0, 128)] + r_adA[pl.ds(0, 128)]
    pltpu.sync_copy(r_asA.at[pl.ds(0, 128)], s_den.at[pl.ds(rr, 128)])
  plsc.subcore_barrier()
  base = wid * (n_chunks * Ck)

  def issue_a(eoff):
    pltpu.sync_copy(src_h.at[pl.ds(eoff, Ck)], v_srcA)
    pltpu.sync_copy(dst_h.at[pl.ds(eoff, Ck)], v_dstA)
    pltpu.async_copy(s_as.at[v_srcA], r_asA, sA2)
    pltpu.async_copy(xp2_h.at[v_srcA], r_xpA, sA5)
    pltpu.async_copy(s_ad.at[v_dstA], r_adA, sA3)
    pltpu.async_copy(s_m.at[v_dstA], r_mA, sA4)
    pltpu.async_copy(s_den.at[v_dstA], r_denA, sA0)

  def wait_a():
    pltpu.make_async_copy(s_as.at[v_srcA], r_asA, sA2).wait()
    pltpu.make_async_copy(xp2_h.at[v_srcA], r_xpA, sA5).wait()
    pltpu.make_async_copy(s_ad.at[v_dstA], r_adA, sA3).wait()
    pltpu.make_async_copy(s_m.at[v_dstA], r_mA, sA4).wait()
    pltpu.make_async_copy(s_den.at[v_dstA], r_denA, sA0).wait()

  def compute(r_as, r_ad, r_m, r_den, r_xp):
    def cblk(j, c2):
      sl = pl.ds(j * 128, 128)
      a = r_as[sl] + r_ad[sl]
      ex = jnp.exp(_lrelu(a, 0.2) - r_m[sl])
      r_den[sl] = ex / (r_den[sl] + 1e-16)
      return c2

    lax.fori_loop(0, Ck // 128, cblk, 0)

    def blk(j, c2):
      b0 = j * 16
      cfb = r_den[pl.ds(b0, 16)]
      for k in range(16):
        r_xp[b0 + k, :] = r_xp[b0 + k, :] * cfb[k]
      return c2

    lax.fori_loop(0, Ck // 16, blk, 0)

  issue_a(base)

  def pair(i, carry):
    ea = base + 2 * i * Ck
    eb = ea + Ck
    en = ea + 2 * Ck
    db0 = pltpu.async_copy(src_h.at[pl.ds(eb, Ck)], v_srcB, sB0)
    db1 = pltpu.async_copy(dst_h.at[pl.ds(eb, Ck)], v_dstB, sB1)
    wait_a()
    db0.wait()
    gb2 = pltpu.async_copy(s_as.at[v_srcB], r_asB, sB2)
    gb5 = pltpu.async_copy(xp2_h.at[v_srcB], r_xpB, sB5)
    db1.wait()
    gb3 = pltpu.async_copy(s_ad.at[v_dstB], r_adB, sB3)
    gb4 = pltpu.async_copy(s_m.at[v_dstB], r_mB, sB4)
    gbd = pltpu.async_copy(s_den.at[v_dstB], r_denB, sB0)
    compute(r_asA, r_adA, r_mA, r_denA, r_xpA)
    sca = pltpu.async_copy(r_xpA, s_out.at[v_dstA], sSA, add=True)
    gb2.wait()
    gb5.wait()
    gb3.wait()
    gb4.wait()
    gbd.wait()
    compute(r_asB, r_adB, r_mB, r_denB, r_xpB)
    scb = pltpu.async_copy(r_xpB, s_out.at[v_dstB], sSB, add=True)
    sca.wait()
    issue_a(en)
    scb.wait()
    return carry

  lax.fori_loop(0, n_chunks // 2, pair, 0)
  wait_a()
  plsc.subcore_barrier()
  pltpu.sync_copy(s_out.at[pl.ds(r0, ROWS_PER_SC)],
                  out_o.at[pl.ds(cid * NP + r0, ROWS_PER_SC)])


@jax.jit
def kernel(inputs, edge_index, W1, att_src1, att_dst1, b1,
           W2, att_src2, att_dst2, b2):
  f32 = jnp.float32
  C1 = 768
  C2 = 384
  Et = edge_index.shape[1] + N
  W = -(-Et // (NW * C1)) * C1
  n1 = W // C1
  n2 = W // C2
  EP = NW * W
  pad = EP + C1 - Et
  loops = jnp.arange(N, dtype=jnp.int32)
  pad_src = (jnp.arange(pad, dtype=jnp.int32) * 37) % N
  pad_dst = N + (jnp.arange(pad, dtype=jnp.int32) % (NP - N))
  src = jnp.concatenate([edge_index[0], loops, pad_src])
  dst = jnp.concatenate([edge_index[1], loops, pad_dst])

  eye8 = jnp.eye(H1, dtype=f32)
  asrc_w = (att_src1[:, :, None] * eye8[:, None, :]).reshape(F1, H1)
  adst_w = (att_dst1[:, :, None] * eye8[:, None, :]).reshape(F1, H1)
  perm = (jnp.arange(F1) % HID) * H1 + jnp.arange(F1) // HID
  P = jax.nn.one_hot(perm, F1, dtype=f32)
  z8 = jnp.zeros((NP, H1), f32)
  z16 = jnp.zeros((NP, OUT), f32)
  z64 = jnp.zeros((NP, F1), f32)

  asrc, adst, m, xpp = pl.pallas_call(
      _tc1_body,
      out_shape=[
          jax.ShapeDtypeStruct((NP, H1), f32),
          jax.ShapeDtypeStruct((NP, H1), f32),
          jax.ShapeDtypeStruct((NP, H1), f32),
          jax.ShapeDtypeStruct((NP, F1), jnp.bfloat16),
      ],
  )(inputs, W1, asrc_w, adst_w, P)

  mesh = plsc.VectorSubcoreMesh(core_axis_name="c", subcore_axis_name="s")
  sc_params = pltpu.CompilerParams(use_tc_tiling_on_sc=False)
  sems = [pltpu.SemaphoreType.DMA] * 5

  den_part, ex_all = pl.kernel(
      functools.partial(_sc_pass1_body, C1, n1, H1, True),
      out_type=[
          jax.ShapeDtypeStruct((NC * NP, H1), f32),
          jax.ShapeDtypeStruct((EP + C1, H1), f32),
      ],
      mesh=mesh,
      compiler_params=sc_params,
      scratch_types=[
          pltpu.VMEM_SHARED((NP, H1), f32),
          pltpu.VMEM_SHARED((NP, H1), f32),
          pltpu.VMEM_SHARED((NP, H1), f32),
          pltpu.VMEM_SHARED((NP, H1), f32),
          pltpu.VMEM((C1,), jnp.int32),
          pltpu.VMEM((C1,), jnp.int32),
          pltpu.VMEM((C1, H1), f32),
          pltpu.VMEM((C1, H1), f32),
          pltpu.VMEM((C1, H1), f32),
          pltpu.VMEM((C1, H1), f32),
      ] + sems,
  )(asrc, adst, m, src, dst, z8)

  dbl = lambda shapes: shapes + shapes
  sems6 = [pltpu.SemaphoreType.DMA] * 6
  sems7 = [pltpu.SemaphoreType.DMA] * 7
  C2a = 256
  n2a = W // C2a
  out1_part = pl.kernel(
      functools.partial(_sc_pass2_body, C2a, n2a),
      out_type=jax.ShapeDtypeStruct((NC * NP, F1), f32),
      mesh=mesh,
      compiler_params=sc_params,
      scratch_types=[
          pltpu.VMEM_SHARED((NP, H1), f32),
          pltpu.VMEM_SHARED((NP, F1), f32),
      ] + dbl([
          pltpu.VMEM((C2a,), jnp.int32),
          pltpu.VMEM((C2a,), jnp.int32),
          pltpu.VMEM((C2a, H1), f32),
          pltpu.VMEM((C2a, H1), f32),
          pltpu.VMEM((C2a, F1), jnp.bfloat16),
          pltpu.VMEM((C2a, F1), f32),
      ]) + sems6 + sems6,
  )(den_part, xpp, ex_all, src, dst, z64)

  xp2, as2, ad2, m2 = pl.pallas_call(
      _tc2_body,
      out_shape=[
          jax.ShapeDtypeStruct((NP, OUT), f32),
          jax.ShapeDtypeStruct((1, NP), f32),
          jax.ShapeDtypeStruct((1, NP), f32),
          jax.ShapeDtypeStruct((1, NP), f32),
      ],
  )(out1_part, b1.reshape(1, F1), W2, att_src2, att_dst2, P)
  as2 = as2.reshape(NP)
  ad2 = ad2.reshape(NP)
  m2 = m2.reshape(NP)
  z1 = jnp.zeros((NP,), f32)

  den2_part = pl.kernel(
      functools.partial(_sc2_pass1_body, C1, n1),
      out_type=jax.ShapeDtypeStruct((NC * NP,), f32),
      mesh=mesh,
      compiler_params=sc_params,
      scratch_types=[
          pltpu.VMEM_SHARED((NP,), f32),
          pltpu.VMEM_SHARED((NP,), f32),
          pltpu.VMEM_SHARED((NP,), f32),
          pltpu.VMEM_SHARED((NP,), f32),
          pltpu.VMEM((C1,), jnp.int32),
          pltpu.VMEM((C1,), jnp.int32),
          pltpu.VMEM((C1,), f32),
          pltpu.VMEM((C1,), f32),
          pltpu.VMEM((C1,), f32),
          pltpu.VMEM((C1,), f32),
      ] + sems,
  )(as2, ad2, m2, src, dst, z1)

  out2_part = pl.kernel(
      functools.partial(_sc2_pass2_body, C2, n2),
      out_type=jax.ShapeDtypeStruct((NC * NP, OUT), f32),
      mesh=mesh,
      compiler_params=sc_params,
      scratch_types=[
          pltpu.VMEM_SHARED((NP,), f32),
          pltpu.VMEM_SHARED((NP,), f32),
          pltpu.VMEM_SHARED((NP,), f32),
          pltpu.VMEM_SHARED((NP,), f32),
          pltpu.VMEM_SHARED((NP, OUT), f32),
      ] + dbl([
          pltpu.VMEM((C2,), jnp.int32),
          pltpu.VMEM((C2,), jnp.int32),
          pltpu.VMEM((C2,), f32),
          pltpu.VMEM((C2,), f32),
          pltpu.VMEM((C2,), f32),
          pltpu.VMEM((C2,), f32),
          pltpu.VMEM((C2, OUT), f32),
      ]) + sems7 + sems7,
  )(as2, ad2, m2, den2_part, xp2, src, dst, z16)

  out = pl.pallas_call(
      _tc3_body,
      out_shape=jax.ShapeDtypeStruct((N, OUT), f32),
  )(out2_part, b2.reshape(1, OUT))
  return out


# speedup vs baseline: 1.0246x; 1.0246x over previous
import functools

import jax
import jax.numpy as jnp
from jax import lax
from jax.experimental import pallas as pl
from jax.experimental.pallas import tpu as pltpu
from jax.experimental.pallas import tpu_sc as plsc

N = 10000
NP = 10240
IN = 128
H1 = 8
HID = 8
F1 = H1 * HID
OUT = 16
NC = 2
NS = 16
NW = NC * NS
ROWS_PER_SC = NP // NS
C = 512


def _lrelu(x, slope):
  return jnp.where(x > 0, x, x * slope)


def _tc1_body(x_ref, w1_ref, asrc_w_ref, adst_w_ref, p_ref,
              asrc_o, adst_o, m_o, xpp_o):
  x = x_ref[...]
  xp = lax.dot_general(x, w1_ref[...], (((1,), (1,)), ((), ())),
                       preferred_element_type=jnp.float32)
  asrc = jnp.dot(xp, asrc_w_ref[...], preferred_element_type=jnp.float32)
  adst = jnp.dot(xp, adst_w_ref[...], preferred_element_type=jnp.float32)
  gmax = jnp.max(asrc, axis=0, keepdims=True)
  m = _lrelu(gmax + adst, 0.2)
  xpp = jnp.dot(xp, p_ref[...], preferred_element_type=jnp.float32)
  zpad8 = jnp.zeros((NP - N, H1), jnp.float32)
  asrc_o[...] = jnp.concatenate([asrc, zpad8], axis=0)
  adst_o[...] = jnp.concatenate([adst, zpad8], axis=0)
  m_o[...] = jnp.concatenate([m, zpad8], axis=0)
  xpp_o[...] = jnp.concatenate([xpp, jnp.zeros((NP - N, F1), jnp.float32)],
                               axis=0)


def _tc2_body(outp_ref, b1_ref, w2_ref, as2_ref, ad2_ref, p_ref,
              xp2_o, as2_o, ad2_o, m2_o):
  s = outp_ref[0:NP, :] + outp_ref[NP:2 * NP, :]
  o1 = lax.dot_general(s, p_ref[...], (((1,), (1,)), ((), ())),
                       preferred_element_type=jnp.float32)
  h = _lrelu(o1 + b1_ref[...], 0.01)
  row = lax.broadcasted_iota(jnp.int32, (NP, 1), 0)
  h = jnp.where(row < N, h, 0.0)
  xp2 = lax.dot_general(h, w2_ref[...], (((1,), (1,)), ((), ())),
                        preferred_element_type=jnp.float32)
  asrc2 = lax.dot_general(as2_ref[...], xp2, (((1,), (1,)), ((), ())),
                          preferred_element_type=jnp.float32)
  adst2 = lax.dot_general(ad2_ref[...], xp2, (((1,), (1,)), ((), ())),
                          preferred_element_type=jnp.float32)
  gmax2 = jnp.max(asrc2)
  m2 = _lrelu(gmax2 + adst2, 0.2)
  xp2_o[...] = xp2
  as2_o[...] = asrc2
  ad2_o[...] = adst2
  m2_o[...] = m2


def _tc3_body(outp_ref, b2_ref, o_ref):
  o_ref[...] = (outp_ref[0:N, :] + outp_ref[NP:NP + N, :]) + b2_ref[...]


def _stage_slice(hbm_ref, spmem_ref, sid):
  r0 = sid * ROWS_PER_SC
  pltpu.sync_copy(hbm_ref.at[pl.ds(r0, ROWS_PER_SC)],
                  spmem_ref.at[pl.ds(r0, ROWS_PER_SC)])


def _combine_den(den_h, s_den, sid, buf_a, buf_b, F):
  r0 = sid * ROWS_PER_SC
  for i in range(ROWS_PER_SC // 128):
    rr = r0 + i * 128
    pltpu.sync_copy(den_h.at[pl.ds(rr, 128)], buf_a.at[pl.ds(0, 128)])
    pltpu.sync_copy(den_h.at[pl.ds(NP + rr, 128)], buf_b.at[pl.ds(0, 128)])

    def dblk(j, c2):
      sl = pl.ds(j * (128 // F), 128 // F)
      buf_a[sl, :] = buf_a[sl, :] + buf_b[sl, :]
      return c2

    lax.fori_loop(0, 128 // (128 // F), dblk, 0)
    pltpu.sync_copy(buf_a.at[pl.ds(0, 128)], s_den.at[pl.ds(rr, 128)])
  plsc.subcore_barrier()


def _sc_pass1_body(Ck, n_chunks, F, write_ex, asrc_h, adst_h, m_h, src_h,
                   dst_h, z_h,
                   *refs):
  if write_ex:
    den_o, ex_o = refs[0], refs[1]
    rest = refs[2:]
  else:
    den_o = refs[0]
    rest = refs[1:]
  (s_asrc, s_adst, s_m, s_den,
   v_src, v_dst, r_asrc, r_adst, r_m, r_ex,
   sem0, sem1, sem2, sem3, sem4) = rest
  cid = lax.axis_index("c")
  sid = lax.axis_index("s")
  wid = sid * NC + cid
  _stage_slice(asrc_h, s_asrc, sid)
  _stage_slice(adst_h, s_adst, sid)
  _stage_slice(m_h, s_m, sid)
  _stage_slice(z_h, s_den, sid)
  plsc.subcore_barrier()
  base = wid * (n_chunks * Ck)
  B = 128 // F

  def chunk(ch, carry):
    eoff = base + ch * Ck
    d0 = pltpu.async_copy(src_h.at[pl.ds(eoff, Ck)], v_src, sem0)
    d1 = pltpu.async_copy(dst_h.at[pl.ds(eoff, Ck)], v_dst, sem1)
    d0.wait()
    d2 = pltpu.async_copy(s_asrc.at[v_src], r_asrc, sem2)
    d1.wait()
    d3 = pltpu.async_copy(s_adst.at[v_dst], r_adst, sem3)
    d4 = pltpu.async_copy(s_m.at[v_dst], r_m, sem4)
    d2.wait()
    d3.wait()
    d4.wait()

    def blk(j, c2):
      sl = pl.ds(j * B, B)
      a = r_asrc[sl, :] + r_adst[sl, :]
      r_ex[sl, :] = jnp.exp(_lrelu(a, 0.2) - r_m[sl, :])
      return c2

    lax.fori_loop(0, Ck // B, blk, 0)
    pltpu.sync_copy(r_ex, s_den.at[v_dst], add=True)
    if write_ex:
      pltpu.sync_copy(r_ex, ex_o.at[pl.ds(eoff, Ck)])
    return carry

  lax.fori_loop(0, n_chunks, chunk, 0)
  plsc.subcore_barrier()
  r0 = sid * ROWS_PER_SC
  pltpu.sync_copy(s_den.at[pl.ds(r0, ROWS_PER_SC)],
                  den_o.at[pl.ds(cid * NP + r0, ROWS_PER_SC)])


def _sc_pass2_body(Ck, n_chunks, den_h, xpp_h, ex_h, src_h, dst_h, z64_h,
                   out_o,
                   s_den, s_out,
                   v_srcA, v_dstA, r_exA, r_denA, r_xpA,
                   v_srcB, v_dstB, r_exB, r_denB, r_xpB,
                   sA0, sA1, sA2, sA3, sA4, sSA,
                   sB0, sB1, sB2, sB3, sB4, sSB):
  cid = lax.axis_index("c")
  sid = lax.axis_index("s")
  wid = sid * NC + cid
  r0 = sid * ROWS_PER_SC
  _stage_slice(z64_h, s_out, sid)
  _combine_den(den_h, s_den, sid, r_exA, r_denA, H1)
  base = wid * (n_chunks * Ck)

  def issue_a(eoff):
    pltpu.sync_copy(src_h.at[pl.ds(eoff, Ck)], v_srcA)
    pltpu.sync_copy(dst_h.at[pl.ds(eoff, Ck)], v_dstA)
    pltpu.async_copy(xpp_h.at[v_srcA], r_xpA, sA2)
    pltpu.async_copy(ex_h.at[pl.ds(eoff, Ck)], r_exA, sA4)
    pltpu.async_copy(s_den.at[v_dstA], r_denA, sA3)

  def wait_a(eoff):
    pltpu.make_async_copy(xpp_h.at[v_srcA], r_xpA, sA2).wait()
    pltpu.make_async_copy(ex_h.at[pl.ds(eoff, Ck)], r_exA, sA4).wait()
    pltpu.make_async_copy(s_den.at[v_dstA], r_denA, sA3).wait()

  def compute(r_ex, r_den, r_xp):
    def blk(j, c2):
      sl = pl.ds(j * 16, 16)
      cf = r_ex[sl, :] / (r_den[sl, :] + 1e-16)
      for c in range(HID):
        csl = pl.ds(c * H1, H1)
        r_xp[sl, csl] = r_xp[sl, csl] * cf
      return c2

    lax.fori_loop(0, Ck // 16, blk, 0)

  issue_a(base)

  def pair(i, carry):
    ea = base + 2 * i * Ck
    eb = ea + Ck
    en = ea + 2 * Ck
    db0 = pltpu.async_copy(src_h.at[pl.ds(eb, Ck)], v_srcB, sB0)
    db1 = pltpu.async_copy(dst_h.at[pl.ds(eb, Ck)], v_dstB, sB1)
    wait_a(ea)
    db0.wait()
    gb2 = pltpu.async_copy(xpp_h.at[v_srcB], r_xpB, sB2)
    gb4 = pltpu.async_copy(ex_h.at[pl.ds(eb, Ck)], r_exB, sB4)
    db1.wait()
    gb3 = pltpu.async_copy(s_den.at[v_dstB], r_denB, sB3)
    compute(r_exA, r_denA, r_xpA)
    sca = pltpu.async_copy(r_xpA, s_out.at[v_dstA], sSA, add=True)
    gb2.wait()
    gb3.wait()
    gb4.wait()
    compute(r_exB, r_denB, r_xpB)
    scb = pltpu.async_copy(r_xpB, s_out.at[v_dstB], sSB, add=True)
    sca.wait()
    issue_a(en)
    scb.wait()
    return carry

  lax.fori_loop(0, n_chunks // 2, pair, 0)
  wait_a(base + n_chunks * Ck)
  plsc.subcore_barrier()
  pltpu.sync_copy(s_out.at[pl.ds(r0, ROWS_PER_SC)],
                  out_o.at[pl.ds(cid * NP + r0, ROWS_PER_SC)])


def _sc2_pass1_body(Ck, n_chunks, as2_h, ad2_h, m2_h, src_h, dst_h, z1_h,
                    den_o,
                    s_as, s_ad, s_m, s_den,
                    v_src, v_dst, r_as, r_ad, r_m, r_ex,
                    sem0, sem1, sem2, sem3, sem4):
  cid = lax.axis_index("c")
  sid = lax.axis_index("s")
  wid = sid * NC + cid
  _stage_slice(as2_h, s_as, sid)
  _stage_slice(ad2_h, s_ad, sid)
  _stage_slice(m2_h, s_m, sid)
  _stage_slice(z1_h, s_den, sid)
  plsc.subcore_barrier()
  base = wid * (n_chunks * Ck)

  def chunk(ch, carry):
    eoff = base + ch * Ck
    d0 = pltpu.async_copy(src_h.at[pl.ds(eoff, Ck)], v_src, sem0)
    d1 = pltpu.async_copy(dst_h.at[pl.ds(eoff, Ck)], v_dst, sem1)
    d0.wait()
    d2 = pltpu.async_copy(s_as.at[v_src], r_as, sem2)
    d1.wait()
    d3 = pltpu.async_copy(s_ad.at[v_dst], r_ad, sem3)
    d4 = pltpu.async_copy(s_m.at[v_dst], r_m, sem4)
    d2.wait()
    d3.wait()
    d4.wait()

    def blk(j, c2):
      sl = pl.ds(j * 128, 128)
      a = r_as[sl] + r_ad[sl]
      r_ex[sl] = jnp.exp(_lrelu(a, 0.2) - r_m[sl])
      return c2

    lax.fori_loop(0, Ck // 128, blk, 0)
    pltpu.sync_copy(r_ex, s_den.at[v_dst], add=True)
    return carry

  lax.fori_loop(0, n_chunks, chunk, 0)
  plsc.subcore_barrier()
  r0 = sid * ROWS_PER_SC
  pltpu.sync_copy(s_den.at[pl.ds(r0, ROWS_PER_SC)],
                  den_o.at[pl.ds(cid * NP + r0, ROWS_PER_SC)])


def _sc2_pass2_body(Ck, n_chunks, as2_h, ad2_h, m2_h, den_h, xp2_h, src_h,
                    dst_h, z16_h,
                    out_o,
                    s_as, s_ad, s_m, s_den, s_out,
                    v_srcA, v_dstA, r_asA, r_adA, r_mA, r_denA, r_xpA,
                    v_srcB, v_dstB, r_asB, r_adB, r_mB, r_denB, r_xpB,
                    sA0, sA1, sA2, sA3, sA4, sA5, sSA,
                    sB0, sB1, sB2, sB3, sB4, sB5, sSB):
  cid = lax.axis_index("c")
  sid = lax.axis_index("s")
  wid = sid * NC + cid
  r0 = sid * ROWS_PER_SC
  _stage_slice(as2_h, s_as, sid)
  _stage_slice(ad2_h, s_ad, sid)
  _stage_slice(m2_h, s_m, sid)
  _stage_slice(z16_h, s_out, sid)
  for i in range(ROWS_PER_SC // 128):
    rr = r0 + i * 128
    pltpu.sync_copy(den_h.at[pl.ds(rr, 128)], r_asA.at[pl.ds(0, 128)])
    pltpu.sync_copy(den_h.at[pl.ds(NP + rr, 128)], r_adA.at[pl.ds(0, 128)])
    r_asA[pl.ds(0, 128)] = r_asA[pl.ds(0, 128)] + r_adA[pl.ds(0, 128)]
    pltpu.sync_copy(r_asA.at[pl.ds(0, 128)], s_den.at[pl.ds(rr, 128)])
  plsc.subcore_barrier()
  base = wid * (n_chunks * Ck)

  def issue_a(eoff):
    pltpu.sync_copy(src_h.at[pl.ds(eoff, Ck)], v_srcA)
    pltpu.sync_copy(dst_h.at[pl.ds(eoff, Ck)], v_dstA)
    pltpu.async_copy(s_as.at[v_srcA], r_asA, sA2)
    pltpu.async_copy(xp2_h.at[v_srcA], r_xpA, sA5)
    pltpu.async_copy(s_ad.at[v_dstA], r_adA, sA3)
    pltpu.async_copy(s_m.at[v_dstA], r_mA, sA4)
    pltpu.async_copy(s_den.at[v_dstA], r_denA, sA0)

  def wait_a():
    pltpu.make_async_copy(s_as.at[v_srcA], r_asA, sA2).wait()
    pltpu.make_async_copy(xp2_h.at[v_srcA], r_xpA, sA5).wait()
    pltpu.make_async_copy(s_ad.at[v_dstA], r_adA, sA3).wait()
    pltpu.make_async_copy(s_m.at[v_dstA], r_mA, sA4).wait()
    pltpu.make_async_copy(s_den.at[v_dstA], r_denA, sA0).wait()

  def compute(r_as, r_ad, r_m, r_den, r_xp):
    def cblk(j, c2):
      sl = pl.ds(j * 128, 128)
      a = r_as[sl] + r_ad[sl]
      ex = jnp.exp(_lrelu(a, 0.2) - r_m[sl])
      r_den[sl] = ex / (r_den[sl] + 1e-16)
      return c2

    lax.fori_loop(0, Ck // 128, cblk, 0)

    def blk(j, c2):
      b0 = j * 16
      cfb = r_den[pl.ds(b0, 16)]
      for k in range(16):
        r_xp[b0 + k, :] = r_xp[b0 + k, :] * cfb[k]
      return c2

    lax.fori_loop(0, Ck // 16, blk, 0)

  issue_a(base)

  def pair(i, carry):
    ea = base + 2 * i * Ck
    eb = ea + Ck
    en = ea + 2 * Ck
    db0 = pltpu.async_copy(src_h.at[pl.ds(eb, Ck)], v_srcB, sB0)
    db1 = pltpu.async_copy(dst_h.at[pl.ds(eb, Ck)], v_dstB, sB1)
    wait_a()
    db0.wait()
    gb2 = pltpu.async_copy(s_as.at[v_srcB], r_asB, sB2)
    gb5 = pltpu.async_copy(xp2_h.at[v_srcB], r_xpB, sB5)
    db1.wait()
    gb3 = pltpu.async_copy(s_ad.at[v_dstB], r_adB, sB3)
    gb4 = pltpu.async_copy(s_m.at[v_dstB], r_mB, sB4)
    gbd = pltpu.async_copy(s_den.at[v_dstB], r_denB, sB0)
    compute(r_asA, r_adA, r_mA, r_denA, r_xpA)
    sca = pltpu.async_copy(r_xpA, s_out.at[v_dstA], sSA, add=True)
    gb2.wait()
    gb5.wait()
    gb3.wait()
    gb4.wait()
    gbd.wait()
    compute(r_asB, r_adB, r_mB, r_denB, r_xpB)
    scb = pltpu.async_copy(r_xpB, s_out.at[v_dstB], sSB, add=True)
    sca.wait()
    issue_a(en)
    scb.wait()
    return carry

  lax.fori_loop(0, n_chunks // 2, pair, 0)
  wait_a()
  plsc.subcore_barrier()
  pltpu.sync_copy(s_out.at[pl.ds(r0, ROWS_PER_SC)],
                  out_o.at[pl.ds(cid * NP + r0, ROWS_PER_SC)])


@jax.jit
def kernel(inputs, edge_index, W1, att_src1, att_dst1, b1,
           W2, att_src2, att_dst2, b2):
  f32 = jnp.float32
  C1 = 768
  C2 = 384
  Et = edge_index.shape[1] + N
  W = -(-Et // (NW * C1)) * C1
  n1 = W // C1
  n2 = W // C2
  EP = NW * W
  pad = EP + C1 - Et
  loops = jnp.arange(N, dtype=jnp.int32)
  pad_src = (jnp.arange(pad, dtype=jnp.int32) * 37) % N
  pad_dst = N + (jnp.arange(pad, dtype=jnp.int32) % (NP - N))
  src = jnp.concatenate([edge_index[0], loops, pad_src])
  dst = jnp.concatenate([edge_index[1], loops, pad_dst])

  eye8 = jnp.eye(H1, dtype=f32)
  asrc_w = (att_src1[:, :, None] * eye8[:, None, :]).reshape(F1, H1)
  adst_w = (att_dst1[:, :, None] * eye8[:, None, :]).reshape(F1, H1)
  perm = (jnp.arange(F1) % HID) * H1 + jnp.arange(F1) // HID
  P = jax.nn.one_hot(perm, F1, dtype=f32)
  z8 = jnp.zeros((NP, H1), f32)
  z16 = jnp.zeros((NP, OUT), f32)
  z64 = jnp.zeros((NP, F1), f32)

  asrc, adst, m, xpp = pl.pallas_call(
      _tc1_body,
      out_shape=[
          jax.ShapeDtypeStruct((NP, H1), f32),
          jax.ShapeDtypeStruct((NP, H1), f32),
          jax.ShapeDtypeStruct((NP, H1), f32),
          jax.ShapeDtypeStruct((NP, F1), f32),
      ],
  )(inputs, W1, asrc_w, adst_w, P)

  mesh = plsc.VectorSubcoreMesh(core_axis_name="c", subcore_axis_name="s")
  sc_params = pltpu.CompilerParams(use_tc_tiling_on_sc=False)
  sems = [pltpu.SemaphoreType.DMA] * 5

  den_part, ex_all = pl.kernel(
      functools.partial(_sc_pass1_body, C1, n1, H1, True),
      out_type=[
          jax.ShapeDtypeStruct((NC * NP, H1), f32),
          jax.ShapeDtypeStruct((EP + C1, H1), f32),
      ],
      mesh=mesh,
      compiler_params=sc_params,
      scratch_types=[
          pltpu.VMEM_SHARED((NP, H1), f32),
          pltpu.VMEM_SHARED((NP, H1), f32),
          pltpu.VMEM_SHARED((NP, H1), f32),
          pltpu.VMEM_SHARED((NP, H1), f32),
          pltpu.VMEM((C1,), jnp.int32),
          pltpu.VMEM((C1,), jnp.int32),
          pltpu.VMEM((C1, H1), f32),
          pltpu.VMEM((C1, H1), f32),
          pltpu.VMEM((C1, H1), f32),
          pltpu.VMEM((C1, H1), f32),
      ] + sems,
  )(asrc, adst, m, src, dst, z8)

  dbl = lambda shapes: shapes + shapes
  sems6 = [pltpu.SemaphoreType.DMA] * 6
  sems7 = [pltpu.SemaphoreType.DMA] * 7
  C2a = 384
  n2a = W // C2a
  out1_part = pl.kernel(
      functools.partial(_sc_pass2_body, C2a, n2a),
      out_type=jax.ShapeDtypeStruct((NC * NP, F1), f32),
      mesh=mesh,
      compiler_params=sc_params,
      scratch_types=[
          pltpu.VMEM_SHARED((NP, H1), f32),
          pltpu.VMEM_SHARED((NP, F1), f32),
      ] + dbl([
          pltpu.VMEM((C2a,), jnp.int32),
          pltpu.VMEM((C2a,), jnp.int32),
          pltpu.VMEM((C2a, H1), f32),
          pltpu.VMEM((C2a, H1), f32),
          pltpu.VMEM((C2a, F1), f32),
      ]) + sems6 + sems6,
  )(den_part, xpp, ex_all, src, dst, z64)

  xp2, as2, ad2, m2 = pl.pallas_call(
      _tc2_body,
      out_shape=[
          jax.ShapeDtypeStruct((NP, OUT), f32),
          jax.ShapeDtypeStruct((1, NP), f32),
          jax.ShapeDtypeStruct((1, NP), f32),
          jax.ShapeDtypeStruct((1, NP), f32),
      ],
  )(out1_part, b1.reshape(1, F1), W2, att_src2, att_dst2, P)
  as2 = as2.reshape(NP)
  ad2 = ad2.reshape(NP)
  m2 = m2.reshape(NP)
  z1 = jnp.zeros((NP,), f32)

  den2_part = pl.kernel(
      functools.partial(_sc2_pass1_body, C1, n1),
      out_type=jax.ShapeDtypeStruct((NC * NP,), f32),
      mesh=mesh,
      compiler_params=sc_params,
      scratch_types=[
          pltpu.VMEM_SHARED((NP,), f32),
          pltpu.VMEM_SHARED((NP,), f32),
          pltpu.VMEM_SHARED((NP,), f32),
          pltpu.VMEM_SHARED((NP,), f32),
          pltpu.VMEM((C1,), jnp.int32),
          pltpu.VMEM((C1,), jnp.int32),
          pltpu.VMEM((C1,), f32),
          pltpu.VMEM((C1,), f32),
          pltpu.VMEM((C1,), f32),
          pltpu.VMEM((C1,), f32),
      ] + sems,
  )(as2, ad2, m2, src, dst, z1)

  out2_part = pl.kernel(
      functools.partial(_sc2_pass2_body, C2, n2),
      out_type=jax.ShapeDtypeStruct((NC * NP, OUT), f32),
      mesh=mesh,
      compiler_params=sc_params,
      scratch_types=[
          pltpu.VMEM_SHARED((NP,), f32),
          pltpu.VMEM_SHARED((NP,), f32),
          pltpu.VMEM_SHARED((NP,), f32),
          pltpu.VMEM_SHARED((NP,), f32),
          pltpu.VMEM_SHARED((NP, OUT), f32),
      ] + dbl([
          pltpu.VMEM((C2,), jnp.int32),
          pltpu.VMEM((C2,), jnp.int32),
          pltpu.VMEM((C2,), f32),
          pltpu.VMEM((C2,), f32),
          pltpu.VMEM((C2,), f32),
          pltpu.VMEM((C2,), f32),
          pltpu.VMEM((C2, OUT), f32),
      ]) + sems7 + sems7,
  )(as2, ad2, m2, den2_part, xp2, src, dst, z16)

  out = pl.pallas_call(
      _tc3_body,
      out_shape=jax.ShapeDtypeStruct((N, OUT), f32),
  )(out2_part, b2.reshape(1, OUT))
  return out


# speedup vs baseline: 1.0626x; 1.0371x over previous
import functools

import jax
import jax.numpy as jnp
from jax import lax
from jax.experimental import pallas as pl
from jax.experimental.pallas import tpu as pltpu
from jax.experimental.pallas import tpu_sc as plsc

N = 10000
NP = 10240
IN = 128
H1 = 8
HID = 8
F1 = H1 * HID
OUT = 16
NC = 2
NS = 16
NW = NC * NS
ROWS_PER_SC = NP // NS
C = 512


def _lrelu(x, slope):
  return jnp.where(x > 0, x, x * slope)


def _tc1_body(x_ref, w1_ref, asrc_w_ref, adst_w_ref, p_ref,
              asrc_o, adst_o, m_o, xpp_o):
  x = x_ref[...]
  xp = lax.dot_general(x, w1_ref[...], (((1,), (1,)), ((), ())),
                       preferred_element_type=jnp.float32)
  asrc = jnp.dot(xp, asrc_w_ref[...], preferred_element_type=jnp.float32)
  adst = jnp.dot(xp, adst_w_ref[...], preferred_element_type=jnp.float32)
  gmax = jnp.max(asrc, axis=0, keepdims=True)
  m = _lrelu(gmax + adst, 0.2)
  xpp = jnp.dot(xp, p_ref[...], preferred_element_type=jnp.float32)
  zpad8 = jnp.zeros((NP - N, H1), jnp.float32)
  asrc_o[...] = jnp.concatenate([asrc, zpad8], axis=0)
  adst_o[...] = jnp.concatenate([adst, zpad8], axis=0)
  m_o[...] = jnp.concatenate([m, zpad8], axis=0)
  xpp_o[...] = jnp.concatenate([xpp, jnp.zeros((NP - N, F1), jnp.float32)],
                               axis=0)


def _tc2_body(outp_ref, b1_ref, w2_ref, as2_ref, ad2_ref, p_ref,
              xp2_o, as2_o, ad2_o, m2_o):
  s = outp_ref[0:NP, :] + outp_ref[NP:2 * NP, :]
  o1 = lax.dot_general(s, p_ref[...], (((1,), (1,)), ((), ())),
                       preferred_element_type=jnp.float32)
  h = _lrelu(o1 + b1_ref[...], 0.01)
  row = lax.broadcasted_iota(jnp.int32, (NP, 1), 0)
  h = jnp.where(row < N, h, 0.0)
  xp2 = lax.dot_general(h, w2_ref[...], (((1,), (1,)), ((), ())),
                        preferred_element_type=jnp.float32)
  asrc2 = lax.dot_general(as2_ref[...], xp2, (((1,), (1,)), ((), ())),
                          preferred_element_type=jnp.float32)
  adst2 = lax.dot_general(ad2_ref[...], xp2, (((1,), (1,)), ((), ())),
                          preferred_element_type=jnp.float32)
  gmax2 = jnp.max(asrc2)
  m2 = _lrelu(gmax2 + adst2, 0.2)
  xp2_o[...] = xp2
  as2_o[...] = asrc2
  ad2_o[...] = adst2
  m2_o[...] = m2


def _tc3_body(outp_ref, b2_ref, o_ref):
  o_ref[...] = (outp_ref[0:N, :] + outp_ref[NP:NP + N, :]) + b2_ref[...]


def _stage_slice(hbm_ref, spmem_ref, sid):
  r0 = sid * ROWS_PER_SC
  pltpu.sync_copy(hbm_ref.at[pl.ds(r0, ROWS_PER_SC)],
                  spmem_ref.at[pl.ds(r0, ROWS_PER_SC)])


def _combine_den(den_h, s_den, sid, buf_a, buf_b, F):
  r0 = sid * ROWS_PER_SC
  for i in range(ROWS_PER_SC // 128):
    rr = r0 + i * 128
    pltpu.sync_copy(den_h.at[pl.ds(rr, 128)], buf_a.at[pl.ds(0, 128)])
    pltpu.sync_copy(den_h.at[pl.ds(NP + rr, 128)], buf_b.at[pl.ds(0, 128)])

    def dblk(j, c2):
      sl = pl.ds(j * (128 // F), 128 // F)
      buf_a[sl, :] = buf_a[sl, :] + buf_b[sl, :]
      return c2

    lax.fori_loop(0, 128 // (128 // F), dblk, 0)
    pltpu.sync_copy(buf_a.at[pl.ds(0, 128)], s_den.at[pl.ds(rr, 128)])
  plsc.subcore_barrier()


def _sc_pass1_body(Ck, n_chunks, F, write_ex, asrc_h, adst_h, m_h, src_h,
                   dst_h, z_h,
                   *refs):
  if write_ex:
    den_o, ex_o = refs[0], refs[1]
    rest = refs[2:]
  else:
    den_o = refs[0]
    rest = refs[1:]
  (s_asrc, s_adst, s_m, s_den,
   v_src, v_dst, r_asrc, r_adst, r_m, r_ex,
   sem0, sem1, sem2, sem3, sem4) = rest
  cid = lax.axis_index("c")
  sid = lax.axis_index("s")
  wid = sid * NC + cid
  _stage_slice(asrc_h, s_asrc, sid)
  _stage_slice(adst_h, s_adst, sid)
  _stage_slice(m_h, s_m, sid)
  _stage_slice(z_h, s_den, sid)
  plsc.subcore_barrier()
  base = wid * (n_chunks * Ck)
  B = 128 // F

  def chunk(ch, carry):
    eoff = base + ch * Ck
    d0 = pltpu.async_copy(src_h.at[pl.ds(eoff, Ck)], v_src, sem0)
    d1 = pltpu.async_copy(dst_h.at[pl.ds(eoff, Ck)], v_dst, sem1)
    d0.wait()
    d2 = pltpu.async_copy(s_asrc.at[v_src], r_asrc, sem2)
    d1.wait()
    d3 = pltpu.async_copy(s_adst.at[v_dst], r_adst, sem3)
    d4 = pltpu.async_copy(s_m.at[v_dst], r_m, sem4)
    d2.wait()
    d3.wait()
    d4.wait()

    def blk(j, c2):
      sl = pl.ds(j * B, B)
      a = r_asrc[sl, :] + r_adst[sl, :]
      r_ex[sl, :] = jnp.exp(_lrelu(a, 0.2) - r_m[sl, :])
      return c2

    lax.fori_loop(0, Ck // B, blk, 0)
    pltpu.sync_copy(r_ex, s_den.at[v_dst], add=True)
    if write_ex:
      pltpu.sync_copy(r_ex, ex_o.at[pl.ds(eoff, Ck)])
    return carry

  lax.fori_loop(0, n_chunks, chunk, 0)
  plsc.subcore_barrier()
  r0 = sid * ROWS_PER_SC
  pltpu.sync_copy(s_den.at[pl.ds(r0, ROWS_PER_SC)],
                  den_o.at[pl.ds(cid * NP + r0, ROWS_PER_SC)])


def _sc_pass2_body(Ck, n_chunks, den_h, xpp_h, ex_h, src_h, dst_h, z64_h,
                   out_o,
                   s_den, s_out,
                   v_srcA, v_dstA, r_exA, r_denA, r_xpA,
                   v_srcB, v_dstB, r_exB, r_denB, r_xpB,
                   sA0, sA1, sA2, sA3, sA4, sSA,
                   sB0, sB1, sB2, sB3, sB4, sSB):
  cid = lax.axis_index("c")
  sid = lax.axis_index("s")
  wid = sid * NC + cid
  r0 = sid * ROWS_PER_SC
  _stage_slice(z64_h, s_out, sid)
  _combine_den(den_h, s_den, sid, r_exA, r_denA, H1)
  base = wid * (n_chunks * Ck)

  def issue_a(eoff):
    pltpu.sync_copy(src_h.at[pl.ds(eoff, Ck)], v_srcA)
    pltpu.sync_copy(dst_h.at[pl.ds(eoff, Ck)], v_dstA)
    pltpu.async_copy(xpp_h.at[v_srcA], r_xpA, sA2)
    pltpu.async_copy(ex_h.at[pl.ds(eoff, Ck)], r_exA, sA4)
    pltpu.async_copy(s_den.at[v_dstA], r_denA, sA3)

  def wait_a(eoff):
    pltpu.make_async_copy(xpp_h.at[v_srcA], r_xpA, sA2).wait()
    pltpu.make_async_copy(ex_h.at[pl.ds(eoff, Ck)], r_exA, sA4).wait()
    pltpu.make_async_copy(s_den.at[v_dstA], r_denA, sA3).wait()

  def compute(r_ex, r_den, r_xp):
    def blk(j, c2):
      sl = pl.ds(j * 16, 16)
      cf = r_ex[sl, :] / (r_den[sl, :] + 1e-16)
      for c in range(HID):
        csl = pl.ds(c * H1, H1)
        r_xp[sl, csl] = r_xp[sl, csl] * cf
      return c2

    lax.fori_loop(0, Ck // 16, blk, 0)

  issue_a(base)

  def pair(i, carry):
    ea = base + 2 * i * Ck
    eb = ea + Ck
    en = ea + 2 * Ck
    db0 = pltpu.async_copy(src_h.at[pl.ds(eb, Ck)], v_srcB, sB0)
    db1 = pltpu.async_copy(dst_h.at[pl.ds(eb, Ck)], v_dstB, sB1)
    wait_a(ea)
    db0.wait()
    gb2 = pltpu.async_copy(xpp_h.at[v_srcB], r_xpB, sB2)
    gb4 = pltpu.async_copy(ex_h.at[pl.ds(eb, Ck)], r_exB, sB4)
    db1.wait()
    gb3 = pltpu.async_copy(s_den.at[v_dstB], r_denB, sB3)
    compute(r_exA, r_denA, r_xpA)
    sca = pltpu.async_copy(r_xpA, s_out.at[v_dstA], sSA, add=True)
    gb2.wait()
    gb3.wait()
    gb4.wait()
    compute(r_exB, r_denB, r_xpB)
    scb = pltpu.async_copy(r_xpB, s_out.at[v_dstB], sSB, add=True)
    sca.wait()
    issue_a(en)
    scb.wait()
    return carry

  lax.fori_loop(0, n_chunks // 2, pair, 0)
  wait_a(base + n_chunks * Ck)
  plsc.subcore_barrier()
  pltpu.sync_copy(s_out.at[pl.ds(r0, ROWS_PER_SC)],
                  out_o.at[pl.ds(cid * NP + r0, ROWS_PER_SC)])


def _sc2_pass1_body(Ck, n_chunks, as2_h, ad2_h, m2_h, src_h, dst_h, z1_h,
                    den_o,
                    s_as, s_ad, s_m, s_den,
                    v_src, v_dst, r_as, r_ad, r_m, r_ex,
                    sem0, sem1, sem2, sem3, sem4):
  cid = lax.axis_index("c")
  sid = lax.axis_index("s")
  wid = sid * NC + cid
  _stage_slice(as2_h, s_as, sid)
  _stage_slice(ad2_h, s_ad, sid)
  _stage_slice(m2_h, s_m, sid)
  _stage_slice(z1_h, s_den, sid)
  plsc.subcore_barrier()
  base = wid * (n_chunks * Ck)

  def chunk(ch, carry):
    eoff = base + ch * Ck
    d0 = pltpu.async_copy(src_h.at[pl.ds(eoff, Ck)], v_src, sem0)
    d1 = pltpu.async_copy(dst_h.at[pl.ds(eoff, Ck)], v_dst, sem1)
    d0.wait()
    d2 = pltpu.async_copy(s_as.at[v_src], r_as, sem2)
    d1.wait()
    d3 = pltpu.async_copy(s_ad.at[v_dst], r_ad, sem3)
    d4 = pltpu.async_copy(s_m.at[v_dst], r_m, sem4)
    d2.wait()
    d3.wait()
    d4.wait()

    def blk(j, c2):
      sl = pl.ds(j * 128, 128)
      a = r_as[sl] + r_ad[sl]
      r_ex[sl] = jnp.exp(_lrelu(a, 0.2) - r_m[sl])
      return c2

    lax.fori_loop(0, Ck // 128, blk, 0)
    pltpu.sync_copy(r_ex, s_den.at[v_dst], add=True)
    return carry

  lax.fori_loop(0, n_chunks, chunk, 0)
  plsc.subcore_barrier()
  r0 = sid * ROWS_PER_SC
  pltpu.sync_copy(s_den.at[pl.ds(r0, ROWS_PER_SC)],
                  den_o.at[pl.ds(cid * NP + r0, ROWS_PER_SC)])


def _sc2_pass2_body(Ck, n_chunks, as2_h, ad2_h, m2_h, den_h, xp2_h, src_h,
                    dst_h, z16_h,
                    out_o,
                    s_as, s_ad, s_m, s_den, s_out,
                    v_srcA, v_dstA, r_asA, r_adA, r_mA, r_denA, r_xpA,
                    v_srcB, v_dstB, r_asB, r_adB, r_mB, r_denB, r_xpB,
                    sA0, sA1, sA2, sA3, sA4, sA5, sSA,
                    sB0, sB1, sB2, sB3, sB4, sB5, sSB):
  cid = lax.axis_index("c")
  sid = lax.axis_index("s")
  wid = sid * NC + cid
  r0 = sid * ROWS_PER_SC
  _stage_slice(as2_h, s_as, sid)
  _stage_slice(ad2_h, s_ad, sid)
  _stage_slice(m2_h, s_m, sid)
  _stage_slice(z16_h, s_out, sid)
  for i in range(ROWS_PER_SC // 128):
    rr = r0 + i * 128
    pltpu.sync_copy(den_h.at[pl.ds(rr, 128)], r_asA.at[pl.ds(0, 128)])
    pltpu.sync_copy(den_h.at[pl.ds(NP + rr, 128)], r_adA.at[pl.ds(0, 128)])
    r_asA[pl.ds(0, 128)] = r_asA[pl.ds(0, 128)] + r_adA[pl.ds(0, 128)]
    pltpu.sync_copy(r_asA.at[pl.ds(0, 128)], s_den.at[pl.ds(rr, 128)])
  plsc.subcore_barrier()
  base = wid * (n_chunks * Ck)

  def issue_a(eoff):
    pltpu.sync_copy(src_h.at[pl.ds(eoff, Ck)], v_srcA)
    pltpu.sync_copy(dst_h.at[pl.ds(eoff, Ck)], v_dstA)
    pltpu.async_copy(s_as.at[v_srcA], r_asA, sA2)
    pltpu.async_copy(xp2_h.at[v_srcA], r_xpA, sA5)
    pltpu.async_copy(s_ad.at[v_dstA], r_adA, sA3)
    pltpu.async_copy(s_m.at[v_dstA], r_mA, sA4)
    pltpu.async_copy(s_den.at[v_dstA], r_denA, sA0)

  def wait_a():
    pltpu.make_async_copy(s_as.at[v_srcA], r_asA, sA2).wait()
    pltpu.make_async_copy(xp2_h.at[v_srcA], r_xpA, sA5).wait()
    pltpu.make_async_copy(s_ad.at[v_dstA], r_adA, sA3).wait()
    pltpu.make_async_copy(s_m.at[v_dstA], r_mA, sA4).wait()
    pltpu.make_async_copy(s_den.at[v_dstA], r_denA, sA0).wait()

  def compute(r_as, r_ad, r_m, r_den, r_xp):
    def cblk(j, c2):
      sl = pl.ds(j * 128, 128)
      a = r_as[sl] + r_ad[sl]
      ex = jnp.exp(_lrelu(a, 0.2) - r_m[sl])
      r_den[sl] = ex / (r_den[sl] + 1e-16)
      return c2

    lax.fori_loop(0, Ck // 128, cblk, 0)

    def blk(j, c2):
      b0 = j * 16
      cfb = r_den[pl.ds(b0, 16)]
      for k in range(16):
        r_xp[b0 + k, :] = r_xp[b0 + k, :] * cfb[k]
      return c2

    lax.fori_loop(0, Ck // 16, blk, 0)

  issue_a(base)

  def pair(i, carry):
    ea = base + 2 * i * Ck
    eb = ea + Ck
    en = ea + 2 * Ck
    db0 = pltpu.async_copy(src_h.at[pl.ds(eb, Ck)], v_srcB, sB0)
    db1 = pltpu.async_copy(dst_h.at[pl.ds(eb, Ck)], v_dstB, sB1)
    wait_a()
    db0.wait()
    gb2 = pltpu.async_copy(s_as.at[v_srcB], r_asB, sB2)
    gb5 = pltpu.async_copy(xp2_h.at[v_srcB], r_xpB, sB5)
    db1.wait()
    gb3 = pltpu.async_copy(s_ad.at[v_dstB], r_adB, sB3)
    gb4 = pltpu.async_copy(s_m.at[v_dstB], r_mB, sB4)
    gbd = pltpu.async_copy(s_den.at[v_dstB], r_denB, sB0)
    compute(r_asA, r_adA, r_mA, r_denA, r_xpA)
    sca = pltpu.async_copy(r_xpA, s_out.at[v_dstA], sSA, add=True)
    gb2.wait()
    gb5.wait()
    gb3.wait()
    gb4.wait()
    gbd.wait()
    compute(r_asB, r_adB, r_mB, r_denB, r_xpB)
    scb = pltpu.async_copy(r_xpB, s_out.at[v_dstB], sSB, add=True)
    sca.wait()
    issue_a(en)
    scb.wait()
    return carry

  lax.fori_loop(0, n_chunks // 2, pair, 0)
  wait_a()
  plsc.subcore_barrier()
  pltpu.sync_copy(s_out.at[pl.ds(r0, ROWS_PER_SC)],
                  out_o.at[pl.ds(cid * NP + r0, ROWS_PER_SC)])


@jax.jit
def kernel(inputs, edge_index, W1, att_src1, att_dst1, b1,
           W2, att_src2, att_dst2, b2):
  f32 = jnp.float32
  C1 = 768
  C2 = 384
  Et = edge_index.shape[1] + N
  W = -(-Et // (NW * C1)) * C1
  n1 = W // C1
  n2 = W // C2
  EP = NW * W
  pad = EP + C1 - Et
  loops = jnp.arange(N, dtype=jnp.int32)
  pad_src = (jnp.arange(pad, dtype=jnp.int32) * 37) % N
  pad_dst = N + (jnp.arange(pad, dtype=jnp.int32) % (NP - N))
  src = jnp.concatenate([edge_index[0], loops, pad_src])
  dst = jnp.concatenate([edge_index[1], loops, pad_dst])

  eye8 = jnp.eye(H1, dtype=f32)
  asrc_w = (att_src1[:, :, None] * eye8[:, None, :]).reshape(F1, H1)
  adst_w = (att_dst1[:, :, None] * eye8[:, None, :]).reshape(F1, H1)
  perm = (jnp.arange(F1) % HID) * H1 + jnp.arange(F1) // HID
  P = jax.nn.one_hot(perm, F1, dtype=f32)
  z8 = jnp.zeros((NP, H1), f32)
  z16 = jnp.zeros((NP, OUT), f32)
  z64 = jnp.zeros((NP, F1), f32)

  asrc, adst, m, xpp = pl.pallas_call(
      _tc1_body,
      out_shape=[
          jax.ShapeDtypeStruct((NP, H1), f32),
          jax.ShapeDtypeStruct((NP, H1), f32),
          jax.ShapeDtypeStruct((NP, H1), f32),
          jax.ShapeDtypeStruct((NP, F1), f32),
      ],
  )(inputs, W1, asrc_w, adst_w, P)

  mesh = plsc.VectorSubcoreMesh(core_axis_name="c", subcore_axis_name="s")
  sc_params = pltpu.CompilerParams(use_tc_tiling_on_sc=False)
  sems = [pltpu.SemaphoreType.DMA] * 5

  den_part, ex_all = pl.kernel(
      functools.partial(_sc_pass1_body, C1, n1, H1, True),
      out_type=[
          jax.ShapeDtypeStruct((NC * NP, H1), f32),
          jax.ShapeDtypeStruct((EP + C1, H1), f32),
      ],
      mesh=mesh,
      compiler_params=sc_params,
      scratch_types=[
          pltpu.VMEM_SHARED((NP, H1), f32),
          pltpu.VMEM_SHARED((NP, H1), f32),
          pltpu.VMEM_SHARED((NP, H1), f32),
          pltpu.VMEM_SHARED((NP, H1), f32),
          pltpu.VMEM((C1,), jnp.int32),
          pltpu.VMEM((C1,), jnp.int32),
          pltpu.VMEM((C1, H1), f32),
          pltpu.VMEM((C1, H1), f32),
          pltpu.VMEM((C1, H1), f32),
          pltpu.VMEM((C1, H1), f32),
      ] + sems,
  )(asrc, adst, m, src, dst, z8)

  dbl = lambda shapes: shapes + shapes
  sems6 = [pltpu.SemaphoreType.DMA] * 6
  sems7 = [pltpu.SemaphoreType.DMA] * 7
  C2a = 384
  n2a = W // C2a
  out1_part = pl.kernel(
      functools.partial(_sc_pass2_body, C2a, n2a),
      out_type=jax.ShapeDtypeStruct((NC * NP, F1), f32),
      mesh=mesh,
      compiler_params=sc_params,
      scratch_types=[
          pltpu.VMEM_SHARED((NP, H1), f32),
          pltpu.VMEM_SHARED((NP, F1), f32),
      ] + dbl([
          pltpu.VMEM((C2a,), jnp.int32),
          pltpu.VMEM((C2a,), jnp.int32),
          pltpu.VMEM((C2a, H1), f32),
          pltpu.VMEM((C2a, H1), f32),
          pltpu.VMEM((C2a, F1), f32),
      ]) + sems6 + sems6,
  )(den_part, xpp, ex_all, src, dst, z64)

  xp2, as2, ad2, m2 = pl.pallas_call(
      _tc2_body,
      out_shape=[
          jax.ShapeDtypeStruct((NP, OUT), f32),
          jax.ShapeDtypeStruct((1, NP), f32),
          jax.ShapeDtypeStruct((1, NP), f32),
          jax.ShapeDtypeStruct((1, NP), f32),
      ],
  )(out1_part, b1.reshape(1, F1), W2, att_src2, att_dst2, P)
  as2 = as2.reshape(NP)
  ad2 = ad2.reshape(NP)
  m2 = m2.reshape(NP)
  z1 = jnp.zeros((NP,), f32)

  den2_part = pl.kernel(
      functools.partial(_sc2_pass1_body, C1, n1),
      out_type=jax.ShapeDtypeStruct((NC * NP,), f32),
      mesh=mesh,
      compiler_params=sc_params,
      scratch_types=[
          pltpu.VMEM_SHARED((NP,), f32),
          pltpu.VMEM_SHARED((NP,), f32),
          pltpu.VMEM_SHARED((NP,), f32),
          pltpu.VMEM_SHARED((NP,), f32),
          pltpu.VMEM((C1,), jnp.int32),
          pltpu.VMEM((C1,), jnp.int32),
          pltpu.VMEM((C1,), f32),
          pltpu.VMEM((C1,), f32),
          pltpu.VMEM((C1,), f32),
          pltpu.VMEM((C1,), f32),
      ] + sems,
  )(as2, ad2, m2, src, dst, z1)

  C2b = 768
  n2b = W // C2b
  out2_part = pl.kernel(
      functools.partial(_sc2_pass2_body, C2b, n2b),
      out_type=jax.ShapeDtypeStruct((NC * NP, OUT), f32),
      mesh=mesh,
      compiler_params=sc_params,
      scratch_types=[
          pltpu.VMEM_SHARED((NP,), f32),
          pltpu.VMEM_SHARED((NP,), f32),
          pltpu.VMEM_SHARED((NP,), f32),
          pltpu.VMEM_SHARED((NP,), f32),
          pltpu.VMEM_SHARED((NP, OUT), f32),
      ] + dbl([
          pltpu.VMEM((C2b,), jnp.int32),
          pltpu.VMEM((C2b,), jnp.int32),
          pltpu.VMEM((C2b,), f32),
          pltpu.VMEM((C2b,), f32),
          pltpu.VMEM((C2b,), f32),
          pltpu.VMEM((C2b,), f32),
          pltpu.VMEM((C2b, OUT), f32),
      ]) + sems7 + sems7,
  )(as2, ad2, m2, den2_part, xp2, src, dst, z16)

  out = pl.pallas_call(
      _tc3_body,
      out_shape=jax.ShapeDtypeStruct((N, OUT), f32),
  )(out2_part, b2.reshape(1, OUT))
  return out
